# baseline (speedup 1.0000x reference)
import functools

import jax
import jax.numpy as jnp
from jax import lax
from jax.experimental import pallas as pl
from jax.experimental.pallas import tpu as pltpu
from jax.experimental.pallas import tpu_sc as plsc

NC = 2
NS = 16
NW = NC * NS
NBUF = 4


@functools.partial(jax.jit, static_argnames=("rows_w", "t", "d"))
def _embed(x, table, rows_w, t, d):
    n_x = x.shape[0]
    mesh = plsc.VectorSubcoreMesh(core_axis_name="c", subcore_axis_name="s")

    @functools.partial(
        pl.kernel,
        out_type=jax.ShapeDtypeStruct((n_x, t, d), jnp.float32),
        mesh=mesh,
        scratch_types=(
            [pltpu.VMEM((rows_w, t), jnp.int32)]
            + [pltpu.VMEM((t, d), jnp.float32) for _ in range(NBUF)]
            + [pltpu.SemaphoreType.DMA for _ in range(2 * NBUF)]
        ),
        compiler_params=pltpu.CompilerParams(use_tc_tiling_on_sc=False),
    )
    def k(x_hbm, table_hbm, out_hbm, idx_v, *bufs_sems):
        bufs = bufs_sems[:NBUF]
        gsems = bufs_sems[NBUF:2 * NBUF]
        osems = bufs_sems[2 * NBUF:]
        wid = lax.axis_index("s") * NC + lax.axis_index("c")
        x0 = wid * rows_w
        pltpu.sync_copy(x_hbm.at[pl.ds(x0, rows_w)], idx_v)

        def fire_gather(y, b):
            pltpu.async_copy(
                table_hbm.at[idx_v.at[y, pl.ds(0, 128)]],
                bufs[b].at[pl.ds(0, 128)],
                gsems[b],
            )
            pltpu.async_copy(
                table_hbm.at[idx_v.at[y, pl.ds(128, t - 128)]],
                bufs[b].at[pl.ds(128, t - 128)],
                gsems[b],
            )

        def wait_gather(b):
            pltpu.make_async_copy(
                table_hbm.at[pl.ds(0, t)], bufs[b], gsems[b]
            ).wait()

        def fire_out(y, b):
            pltpu.async_copy(bufs[b], out_hbm.at[x0 + y], osems[b])

        def wait_out(b):
            pltpu.make_async_copy(bufs[b], out_hbm.at[0], osems[b]).wait()

        fire_gather(0, 0)
        for xr in range(NBUF):
            wait_gather(xr)
            fire_out(xr, xr)
            if xr == NBUF - 1:
                wait_out(0)
            fire_gather(xr + 1, (xr + 1) % NBUF)

        def body(j, c):
            for b in range(NBUF):
                y = j * NBUF + b
                wait_gather(b)
                fire_out(y, b)
                nb = (b + 1) % NBUF
                wait_out(nb)
                fire_gather(y + 1, nb)
            return c

        lax.fori_loop(1, rows_w // NBUF - 1, body, 0)

        for xr in range(rows_w - NBUF, rows_w):
            b = xr % NBUF
            wait_gather(b)
            fire_out(xr, b)
            if xr < rows_w - 1:
                nb = (b + 1) % NBUF
                wait_out(nb)
                fire_gather(xr + 1, nb)
        for b in range(NBUF):
            wait_out(b)

    return k(x, table)


def kernel(x, table):
    b, t = x.shape
    d = table.shape[1]
    return _embed(x, table, b // NW, t, d)


# speedup vs baseline: 1.0361x; 1.0361x over previous
import functools

import jax
import jax.numpy as jnp
from jax import lax
from jax.experimental import pallas as pl
from jax.experimental.pallas import tpu as pltpu
from jax.experimental.pallas import tpu_sc as plsc

NC = 2
NS = 16
NW = NC * NS

K = 128
GPG = 4
ROWS_G = K * GPG


@functools.partial(jax.jit, static_argnames=("b", "d"))
def _gather_rows(xf, table, b, d):
    n_per_w = b // NW
    idx_rows_w = n_per_w // K
    n_groups = n_per_w // ROWS_G
    mesh = plsc.VectorSubcoreMesh(core_axis_name="c", subcore_axis_name="s")

    @functools.partial(
        pl.kernel,
        out_type=jax.ShapeDtypeStruct((b, d), jnp.float32),
        mesh=mesh,
        scratch_types=[
            pltpu.VMEM((idx_rows_w, K), jnp.float32),
            pltpu.VMEM((idx_rows_w, K), jnp.int32),
            pltpu.VMEM((ROWS_G, d), jnp.float32),
            pltpu.VMEM((ROWS_G, d), jnp.float32),
            pltpu.SemaphoreType.DMA,
            pltpu.SemaphoreType.DMA,
            pltpu.SemaphoreType.DMA,
            pltpu.SemaphoreType.DMA,
        ],
        compiler_params=pltpu.CompilerParams(use_tc_tiling_on_sc=False),
    )
    def k(xf_hbm, table_hbm, out_hbm, idxf_v, idx_v, buf0, buf1,
          gsem0, gsem1, osem0, osem1):
        bufs = (buf0, buf1)
        gsems = (gsem0, gsem1)
        osems = (osem0, osem1)
        wid = lax.axis_index("s") * NC + lax.axis_index("c")
        pltpu.sync_copy(xf_hbm.at[pl.ds(wid * idx_rows_w, idx_rows_w)], idxf_v)

        def conv_row(r, c):
            for j in range(K // 16):
                idx_v[r, pl.ds(j * 16, 16)] = idxf_v[
                    r, pl.ds(j * 16, 16)
                ].astype(jnp.int32)
            return c

        lax.fori_loop(0, idx_rows_w, conv_row, 0)

        out_base = wid * n_per_w

        def fire_gather(g, bi):
            for j in range(GPG):
                pltpu.async_copy(
                    table_hbm.at[idx_v.at[g * GPG + j]],
                    bufs[bi].at[pl.ds(j * K, K)],
                    gsems[bi],
                )

        def wait_gather(bi):
            pltpu.make_async_copy(
                table_hbm.at[pl.ds(0, ROWS_G)], bufs[bi], gsems[bi]
            ).wait()

        def fire_out(g, bi):
            pltpu.async_copy(
                bufs[bi],
                out_hbm.at[pl.ds(out_base + g * ROWS_G, ROWS_G)],
                osems[bi],
            )

        def wait_out(bi):
            pltpu.make_async_copy(
                bufs[bi], out_hbm.at[pl.ds(0, ROWS_G)], osems[bi]
            ).wait()

        fire_gather(0, 0)
        wait_gather(0)
        fire_out(0, 0)
        fire_gather(1, 1)

        def body(i, c):
            for bi, off in ((1, 1), (0, 2)):
                g = 2 * i + off
                wait_gather(bi)
                fire_out(g, bi)
                wait_out(1 - bi)
                fire_gather(g + 1, 1 - bi)
            return c

        lax.fori_loop(0, (n_groups - 2) // 2, body, 0)

        wait_gather(1)
        fire_out(n_groups - 1, 1)
        wait_out(0)
        wait_out(1)

    return k(xf, table)


def kernel(x, table):
    b, t = x.shape
    d = table.shape[1]
    xf = x.reshape(-1, K).astype(jnp.float32)
    out = _gather_rows(xf, table, b * t, d)
    return out.reshape(b, t, d)


# speedup vs baseline: 1.3759x; 1.3280x over previous
import functools

import jax
import jax.numpy as jnp
from jax import lax
from jax.experimental import pallas as pl
from jax.experimental.pallas import tpu as pltpu
from jax.experimental.pallas import tpu_sc as plsc

NC = 2
NS = 16
NW = NC * NS

K = 128
GPG = 4
ROWS_G = K * GPG


@functools.partial(jax.jit, static_argnames=("b", "d"))
def _gather_rows(xf, table, b, d):
    n_per_w = b // NW
    n_groups = n_per_w // ROWS_G
    mesh = plsc.VectorSubcoreMesh(core_axis_name="c", subcore_axis_name="s")

    @functools.partial(
        pl.kernel,
        out_type=jax.ShapeDtypeStruct((b, 128), jnp.float32),
        mesh=mesh,
        scratch_types=[
            pltpu.VMEM((n_per_w,), jnp.float32),
            pltpu.VMEM((n_per_w,), jnp.int32),
            pltpu.VMEM((ROWS_G, d), jnp.float32),
            pltpu.VMEM((ROWS_G, d), jnp.float32),
            pltpu.SemaphoreType.DMA,
            pltpu.SemaphoreType.DMA,
            pltpu.SemaphoreType.DMA,
            pltpu.SemaphoreType.DMA,
        ],
        compiler_params=pltpu.CompilerParams(use_tc_tiling_on_sc=False),
    )
    def k(xf_hbm, table_hbm, out_hbm, idxf_v, idx_v, buf0, buf1,
          gsem0, gsem1, osem0, osem1):
        bufs = (buf0, buf1)
        gsems = (gsem0, gsem1)
        osems = (osem0, osem1)
        wid = lax.axis_index("s") * NC + lax.axis_index("c")
        pltpu.sync_copy(xf_hbm.at[pl.ds(wid * n_per_w, n_per_w)], idxf_v)

        def conv_chunk(r, c):
            for j in range(8):
                o = r * 128 + j * 16
                idx_v[pl.ds(o, 16)] = idxf_v[pl.ds(o, 16)].astype(jnp.int32)
            return c

        lax.fori_loop(0, n_per_w // 128, conv_chunk, 0)

        out_base = wid * n_per_w

        def fire_gather(g, bi):
            for j in range(GPG):
                pltpu.async_copy(
                    table_hbm.at[idx_v.at[pl.ds((g * GPG + j) * K, K)]],
                    bufs[bi].at[pl.ds(j * K, K)],
                    gsems[bi],
                )

        def wait_gather(bi):
            pltpu.make_async_copy(
                table_hbm.at[pl.ds(0, ROWS_G)], bufs[bi], gsems[bi]
            ).wait()

        def fire_out(g, bi):
            pltpu.async_copy(
                bufs[bi],
                out_hbm.at[pl.ds(out_base + g * ROWS_G, ROWS_G), pl.ds(0, d)],
                osems[bi],
            )

        def wait_out(bi):
            pltpu.make_async_copy(
                bufs[bi], out_hbm.at[pl.ds(0, ROWS_G), pl.ds(0, d)], osems[bi]
            ).wait()

        fire_gather(0, 0)
        wait_gather(0)
        fire_out(0, 0)
        fire_gather(1, 1)

        def body(i, c):
            for bi, off in ((1, 1), (0, 2)):
                g = 2 * i + off
                wait_gather(bi)
                fire_out(g, bi)
                wait_out(1 - bi)
                fire_gather(g + 1, 1 - bi)
            return c

        lax.fori_loop(0, (n_groups - 2) // 2, body, 0)

        wait_gather(1)
        fire_out(n_groups - 1, 1)
        wait_out(0)
        wait_out(1)

    return k(xf, table)


def kernel(x, table):
    b, t = x.shape
    d = table.shape[1]
    xf = x.reshape(-1).astype(jnp.float32)
    out = _gather_rows(xf, table, b * t, d)
    return out[:, :d].reshape(b, t, d)
